# Initial kernel scaffold; baseline (speedup 1.0000x reference)
import jax
import jax.numpy as jnp
from jax.experimental import pallas as pl


def kernel(x, edge_attr, edge_index, W1a, b1a, W2a, b2a, W1b, b1b, W2b, b2b, gamma_a, beta_a, gamma_b, beta_b):
    raise NotImplementedError("write your pallas kernel here")


# speedup vs baseline: 3.1815x; 3.1815x over previous
import functools

import jax
import jax.numpy as jnp
from jax import lax
from jax.experimental import pallas as pl
from jax.experimental.pallas import tpu as pltpu
from jax.experimental.pallas import tpu_sc as plsc

NC, NS, LANES = 2, 16, 16
NW = NC * NS
CH = 128
ZR = 128
NPAD = 10240
BE = 4000


def _make_sc_gather(N, E, D):
    n_chunks = E // CH
    rows_per_tile = NPAD // NS
    mesh = plsc.VectorSubcoreMesh(core_axis_name="c", subcore_axis_name="s",
                                  num_cores=NC, num_subcores=NS)

    @functools.partial(
        pl.kernel,
        out_type=(jax.ShapeDtypeStruct((E, D), jnp.float32),
                  jax.ShapeDtypeStruct((NC, NPAD, D), jnp.float32)),
        mesh=mesh,
        scratch_types=[
            pltpu.VMEM((CH,), jnp.int32),
            pltpu.VMEM((CH,), jnp.int32),
            pltpu.VMEM((CH, D), jnp.float32),
            pltpu.VMEM((CH, D), jnp.float32),
            pltpu.VMEM_SHARED((NPAD, D), jnp.float32),
            pltpu.SemaphoreType.DMA,
        ],
    )
    def k(src_hbm, dst_hbm, x_hbm, g_hbm, m1_hbm,
          idxs_v, idxd_v, rows_s, rows_d, acc, sem):
        cid = lax.axis_index("c")
        sid = lax.axis_index("s")
        wid = sid * NC + cid
        row0 = sid * rows_per_tile

        def zrow(r, carry):
            for j in range(D // LANES):
                rows_s[r, pl.ds(j * LANES, LANES)] = jnp.zeros((LANES,), jnp.float32)
            return carry
        lax.fori_loop(0, ZR, zrow, 0)
        for t in range(rows_per_tile // ZR):
            pltpu.sync_copy(rows_s, acc.at[pl.ds(row0 + t * ZR, ZR)])
        plsc.subcore_barrier()

        nch = (n_chunks - wid + NW - 1) // NW

        def body(i, carry):
            base = (wid + i * NW) * CH
            pltpu.sync_copy(src_hbm.at[pl.ds(base, CH)], idxs_v)
            pltpu.sync_copy(dst_hbm.at[pl.ds(base, CH)], idxd_v)
            pltpu.async_copy(x_hbm.at[idxs_v], rows_s, sem).wait()
            pltpu.sync_copy(rows_s, acc.at[idxd_v], add=True)
            pltpu.async_copy(x_hbm.at[idxd_v], rows_d, sem).wait()

            def addrow(r, c2):
                for j in range(D // LANES):
                    sl = pl.ds(j * LANES, LANES)
                    rows_d[r, sl] = rows_d[r, sl] + rows_s[r, sl]
                return c2
            lax.fori_loop(0, CH, addrow, 0)
            pltpu.sync_copy(rows_d, g_hbm.at[pl.ds(base, CH)])
            return carry
        lax.fori_loop(0, nch, body, 0)

        plsc.subcore_barrier()
        pltpu.sync_copy(acc.at[pl.ds(row0, rows_per_tile)],
                        m1_hbm.at[cid, pl.ds(row0, rows_per_tile)])
    return k


def _make_sc_norm_scatter(N, E, D):
    CH = 80
    n_chunks = E // CH
    rows_per_tile = NPAD // NS
    mesh = plsc.VectorSubcoreMesh(core_axis_name="c", subcore_axis_name="s",
                                  num_cores=NC, num_subcores=NS)

    @functools.partial(
        pl.kernel,
        out_type=(jax.ShapeDtypeStruct((E, D), jnp.float32),
                  jax.ShapeDtypeStruct((NC, NPAD, D), jnp.float32)),
        mesh=mesh,
        scratch_types=[
            pltpu.VMEM((2, CH), jnp.int32),
            pltpu.VMEM((2, CH, D), jnp.float32),
            pltpu.VMEM((2, CH, D), jnp.float32),
            pltpu.VMEM((D,), jnp.float32),
            pltpu.VMEM((D,), jnp.float32),
            pltpu.VMEM_SHARED((NPAD, D), jnp.float32),
            pltpu.SemaphoreType.DMA,
            pltpu.SemaphoreType.DMA,
            pltpu.SemaphoreType.DMA,
            pltpu.SemaphoreType.DMA,
        ],
    )
    def k(dst_hbm, z_hbm, ea_hbm, sc_hbm, sh_hbm, m1_hbm, e_hbm, m_hbm,
          idx_v, rz, rea, sc_v, sh_v, acc, lsem0, lsem1, ssem0, ssem1):
        cid = lax.axis_index("c")
        sid = lax.axis_index("s")
        wid = sid * NC + cid
        row0 = sid * rows_per_tile
        lsems = (lsem0, lsem1)
        ssems = (ssem0, ssem1)

        pltpu.sync_copy(sc_hbm, sc_v)
        pltpu.sync_copy(sh_hbm, sh_v)
        pltpu.sync_copy(m1_hbm.at[cid, pl.ds(row0, rows_per_tile)],
                        acc.at[pl.ds(row0, rows_per_tile)])
        plsc.subcore_barrier()

        nch = n_chunks // NW

        def start_loads(i, b):
            base = (wid + i * NW) * CH
            pltpu.async_copy(dst_hbm.at[pl.ds(base, CH)], idx_v.at[b],
                             lsems[b])
            pltpu.async_copy(z_hbm.at[pl.ds(base, CH)], rz.at[b], lsems[b])
            pltpu.async_copy(ea_hbm.at[pl.ds(base, CH)], rea.at[b], lsems[b])

        def wait_loads(i, b):
            base = (wid + i * NW) * CH
            pltpu.make_async_copy(dst_hbm.at[pl.ds(base, CH)], idx_v.at[b],
                                  lsems[b]).wait()
            pltpu.make_async_copy(z_hbm.at[pl.ds(base, CH)], rz.at[b],
                                  lsems[b]).wait()
            pltpu.make_async_copy(ea_hbm.at[pl.ds(base, CH)], rea.at[b],
                                  lsems[b]).wait()

        start_loads(0, 0)
        scv = tuple(sc_v[pl.ds(j * LANES, LANES)] for j in range(D // LANES))
        shv = tuple(sh_v[pl.ds(j * LANES, LANES)] for j in range(D // LANES))

        def wait_stores(b):
            pltpu.make_async_copy(rz.at[b], e_hbm.at[pl.ds(0, CH)],
                                  ssems[b]).wait()

        def compute_and_store(i, b):
            def rowfn(r, c2):
                for j in range(D // LANES):
                    sl = pl.ds(j * LANES, LANES)
                    rz[b, r, sl] = (rz[b, r, sl] * scv[j] + shv[j]
                                    + rea[b, r, sl])
                return c2
            lax.fori_loop(0, CH, rowfn, 0)
            base = (wid + i * NW) * CH
            pltpu.async_copy(rz.at[b], e_hbm.at[pl.ds(base, CH)], ssems[b])
            pltpu.sync_copy(rz.at[b], acc.at[idx_v.at[b]], add=True)

        def pair(i2, carry):
            i = 2 * i2

            @pl.when(i2 >= 1)
            def _():
                wait_stores(1)
            start_loads(i + 1, 1)
            wait_loads(i, 0)
            compute_and_store(i, 0)

            wait_stores(0)
            start_loads(i + 2, 0)
            wait_loads(i + 1, 1)
            compute_and_store(i + 1, 1)
            return carry
        lax.fori_loop(0, (nch - 1) // 2, pair, 0)

        wait_stores(1)
        wait_loads(nch - 1, 0)
        compute_and_store(nch - 1, 0)
        wait_stores(0)

        plsc.subcore_barrier()
        pltpu.sync_copy(acc.at[pl.ds(row0, rows_per_tile)],
                        m_hbm.at[cid, pl.ds(row0, rows_per_tile)])
    return k


def _edge_mlp(g, ea, W1, b1, W2, b2):
    E, D = g.shape

    def body(g_ref, ea_ref, w1_ref, b1_ref, w2_ref, b2_ref,
             z_ref, s1_ref, s2_ref):
        i = pl.program_id(0)
        s = g_ref[...] + ea_ref[...]
        h = jnp.maximum(
            jnp.dot(s, w1_ref[...], preferred_element_type=jnp.float32)
            + b1_ref[...], 0.0)
        z = (jnp.dot(h, w2_ref[...], preferred_element_type=jnp.float32)
             + b2_ref[...])
        z_ref[...] = z
        ps1 = jnp.sum(z, axis=0, keepdims=True)
        ps2 = jnp.sum(z * z, axis=0, keepdims=True)

        @pl.when(i == 0)
        def _():
            s1_ref[...] = ps1
            s2_ref[...] = ps2

        @pl.when(i != 0)
        def _():
            s1_ref[...] += ps1
            s2_ref[...] += ps2

    full = lambda i: (0, 0)
    blk = lambda i: (i, 0)
    return pl.pallas_call(
        body,
        grid=(E // BE,),
        in_specs=[
            pl.BlockSpec((BE, D), blk), pl.BlockSpec((BE, D), blk),
            pl.BlockSpec((D, D), full), pl.BlockSpec((1, D), full),
            pl.BlockSpec((D, D), full), pl.BlockSpec((1, D), full),
        ],
        out_specs=[
            pl.BlockSpec((BE, D), blk),
            pl.BlockSpec((1, D), full), pl.BlockSpec((1, D), full),
        ],
        out_shape=[
            jax.ShapeDtypeStruct((E, D), jnp.float32),
            jax.ShapeDtypeStruct((1, D), jnp.float32),
            jax.ShapeDtypeStruct((1, D), jnp.float32),
        ],
    )(g, ea, W1, b1, W2, b2)


def _node_mlp(x, mp, W1, b1, W2, b2, gamma, beta):
    N, D = x.shape

    def body(x_ref, mp_ref, w1_ref, b1_ref, w2_ref, b2_ref,
             gm_ref, bt_ref, h_ref):
        xv = x_ref[...]
        m = mp_ref[0, :N, :] + mp_ref[1, :N, :]
        h1 = jnp.maximum(
            jnp.dot(xv + m, w1_ref[...], preferred_element_type=jnp.float32)
            + b1_ref[...], 0.0)
        u = (jnp.dot(h1, w2_ref[...], preferred_element_type=jnp.float32)
             + b2_ref[...])
        mu = jnp.sum(u, axis=0, keepdims=True) * (1.0 / N)
        d = u - mu
        var = jnp.sum(d * d, axis=0, keepdims=True) * (1.0 / N)
        h_ref[...] = d * lax.rsqrt(var + 1e-5) * gm_ref[...] + bt_ref[...] + xv

    return pl.pallas_call(
        body,
        out_shape=jax.ShapeDtypeStruct((N, D), jnp.float32),
    )(x, mp, W1, b1, W2, b2, gamma, beta)


def kernel(x, edge_attr, edge_index, W1a, b1a, W2a, b2a, W1b, b1b, W2b, b2b,
           gamma_a, beta_a, gamma_b, beta_b):
    N, D = x.shape
    E = edge_attr.shape[0]
    src = edge_index[0]
    dst = edge_index[1]

    g, m1 = _make_sc_gather(N, E, D)(src, dst, x)
    z, s1, s2 = _edge_mlp(g, edge_attr, W1b, b1b.reshape(1, D),
                          W2b, b2b.reshape(1, D))
    mu = s1 * (1.0 / E)
    var = s2 * (1.0 / E) - mu * mu
    scale = gamma_b.reshape(1, D) / jnp.sqrt(var + 1e-5)
    shift = beta_b.reshape(1, D) - mu * scale
    e, mp = _make_sc_norm_scatter(N, E, D)(
        dst, z, edge_attr, scale.reshape(D), shift.reshape(D), m1)
    h = _node_mlp(x, mp, W1a, b1a.reshape(1, D), W2a, b2a.reshape(1, D),
                  gamma_a.reshape(1, D), beta_a.reshape(1, D))
    return (h, e)
